# Initial kernel scaffold; baseline (speedup 1.0000x reference)
import jax
import jax.numpy as jnp
from jax import lax
from jax.experimental import pallas as pl
from jax.experimental.pallas import tpu as pltpu
from jax.experimental.pallas import tpu_sc as plsc

TOPK = 131072
B, L, D = 512, 12, 4096
N = B * L * D
LANES = 16
NC, NS = 2, 16

HI_BUCKETS = 32768
LO_BUCKETS = 65536
HI_CHUNK = HI_BUCKETS // NS
LO_CHUNK = LO_BUCKETS // NS

SC_BLK = 16384
SC_GRID = N // SC_BLK
TC_ROWS = 64


def _sc_mesh():
    return plsc.VectorSubcoreMesh(core_axis_name="c", subcore_axis_name="s")


def _zero_hist(hist_v, nwords):
    zeros16 = jnp.zeros((LANES,), jnp.int32)

    @pl.loop(0, nwords // LANES)
    def _(i):
        hist_v[pl.ds(i * LANES, LANES)] = zeros16


def _merge_hist(c, s, hist_v, tmp_v, acc_v, shared, out_hbm, chunk):
    pltpu.sync_copy(hist_v.at[pl.ds(0, NS * chunk)], shared.at[s])
    plsc.subcore_barrier()
    col = s * chunk
    pltpu.sync_copy(shared.at[0, pl.ds(col, chunk)], acc_v)
    for r in range(1, NS):
        pltpu.sync_copy(shared.at[r, pl.ds(col, chunk)], tmp_v)

        @pl.loop(0, chunk // LANES)
        def _(i):
            sl = pl.ds(i * LANES, LANES)
            acc_v[sl] = acc_v[sl] + tmp_v[sl]

    pltpu.sync_copy(acc_v, out_hbm.at[c, pl.ds(col, chunk)])


def _sc_hist_hi_body(flat_hbm, hist_hbm, hist_v, tmp_v, acc_v, shared):
    c = lax.axis_index("c")
    s = lax.axis_index("s")
    ones16 = jnp.ones((LANES,), jnp.int32)
    lane = lax.iota(jnp.int32, LANES)
    zoff = lane + HI_BUCKETS

    _zero_hist(hist_v, HI_BUCKETS + LANES)

    def body(in_vmem):
        @pl.loop(0, SC_BLK // LANES)
        def _(j):
            v = in_vmem[0, pl.ds(j * LANES, LANES)]
            bits = plsc.bitcast(v, jnp.int32)
            y = jnp.maximum(bits, 0)
            hi = lax.shift_right_logical(y, 16)
            hi2 = jnp.where(hi == 0, zoff, hi)
            plsc.addupdate_scatter(hist_v, [hi2], ones16)

    pltpu.emit_pipeline(
        body,
        grid=(SC_GRID,),
        in_specs=[pl.BlockSpec((1, SC_BLK), lambda i: (i, 0))],
        out_specs=[],
        core_axis_name=("c", "s"),
        dimension_semantics=(pltpu.PARALLEL,),
    )(flat_hbm)

    z = hist_v[pl.ds(HI_BUCKETS, LANES)]
    zsum = jnp.sum(z)
    zvec = jnp.where(lane == 0, zsum, 0)
    sl0 = pl.ds(0, LANES)
    hist_v[sl0] = hist_v[sl0] + zvec

    _merge_hist(c, s, hist_v, tmp_v, acc_v, shared, hist_hbm, HI_CHUNK)


def _sc_hist_lo_body(flat_hbm, hvec_hbm, hist_hbm, hist_v, tmp_v, acc_v,
                     hvec_v, shared):
    c = lax.axis_index("c")
    s = lax.axis_index("s")
    ones16 = jnp.ones((LANES,), jnp.int32)

    _zero_hist(hist_v, LO_BUCKETS)
    pltpu.sync_copy(hvec_hbm, hvec_v)
    hstar = hvec_v[...]

    def body(in_vmem):
        @pl.loop(0, SC_BLK // LANES)
        def _(j):
            v = in_vmem[0, pl.ds(j * LANES, LANES)]
            bits = plsc.bitcast(v, jnp.int32)
            y = jnp.maximum(bits, 0)
            hi = lax.shift_right_logical(y, 16)
            lo = jnp.bitwise_and(y, 0xFFFF)
            mask = hi == hstar
            plsc.addupdate_scatter(hist_v, [lo], ones16, mask=mask)

    pltpu.emit_pipeline(
        body,
        grid=(SC_GRID,),
        in_specs=[pl.BlockSpec((1, SC_BLK), lambda i: (i, 0))],
        out_specs=[],
        core_axis_name=("c", "s"),
        dimension_semantics=(pltpu.PARALLEL,),
    )(flat_hbm)

    _merge_hist(c, s, hist_v, tmp_v, acc_v, shared, hist_hbm, LO_CHUNK)


def _suffix_search(h, nbuckets, k, steps):
    rows = nbuckets // 128
    row = lax.broadcasted_iota(jnp.int32, (rows, 128), 0)
    col = lax.broadcasted_iota(jnp.int32, (rows, 128), 1)
    idx = row * 128 + col

    def cnt_ge(m):
        return jnp.sum(jnp.where(idx >= m, h, 0))

    def step(_, lohi):
        lo, hi = lohi
        mid = (lo + hi) // 2
        ok = cnt_ge(mid) >= k
        return jnp.where(ok, mid, lo), jnp.where(ok, hi, mid)

    lo, _ = lax.fori_loop(0, steps, step, (jnp.int32(0), jnp.int32(nbuckets)))
    above = jnp.sum(jnp.where(idx > lo, h, 0))
    return lo, above


def _tc_find_hi_body(hist_ref, aux_ref):
    h = hist_ref[0] + hist_ref[1]
    hstar, above = _suffix_search(h, HI_BUCKETS, TOPK, 15)
    k2 = TOPK - above
    r = lax.broadcasted_iota(jnp.int32, (8, 128), 0)
    aux_ref[...] = jnp.where(r == 0, hstar, jnp.where(r == 1, k2, 0))


def _tc_mask_body(hist_ref, aux_ref, x_ref, o_ref, t_ref):
    @pl.when(pl.program_id(0) == 0)
    def _():
        hstar = aux_ref[0, 0]
        k2 = aux_ref[1, 0]
        h = hist_ref[0] + hist_ref[1]
        lostar, _ = _suffix_search(h, LO_BUCKETS, k2, 16)
        t_ref[0] = jnp.bitwise_or(lax.shift_left(hstar, 16), lostar)

    t = t_ref[0]
    bits = lax.bitcast_convert_type(x_ref[...], jnp.int32)
    y = jnp.maximum(bits, 0)
    o_ref[...] = lax.bitcast_convert_type(jnp.where(y >= t, y, 0), jnp.float32)


@jax.jit
def kernel(features):
    flat2d = features.reshape(SC_GRID, SC_BLK)

    hist_hi = pl.kernel(
        _sc_hist_hi_body,
        out_type=jax.ShapeDtypeStruct((NC, HI_BUCKETS), jnp.int32),
        mesh=_sc_mesh(),
        scratch_types=[
            pltpu.VMEM((HI_BUCKETS + LANES,), jnp.int32),
            pltpu.VMEM((HI_CHUNK,), jnp.int32),
            pltpu.VMEM((HI_CHUNK,), jnp.int32),
            pltpu.VMEM_SHARED((NS, HI_BUCKETS), jnp.int32),
        ],
    )(flat2d)

    aux = pl.pallas_call(
        _tc_find_hi_body,
        out_shape=jax.ShapeDtypeStruct((8, 128), jnp.int32),
        in_specs=[pl.BlockSpec((NC, HI_BUCKETS // 128, 128),
                               lambda: (0, 0, 0))],
        out_specs=pl.BlockSpec((8, 128), lambda: (0, 0)),
    )(hist_hi.reshape(NC, HI_BUCKETS // 128, 128))

    hvec = aux[0, :LANES]

    hist_lo = pl.kernel(
        _sc_hist_lo_body,
        out_type=jax.ShapeDtypeStruct((NC, LO_BUCKETS), jnp.int32),
        mesh=_sc_mesh(),
        scratch_types=[
            pltpu.VMEM((LO_BUCKETS,), jnp.int32),
            pltpu.VMEM((LO_CHUNK,), jnp.int32),
            pltpu.VMEM((LO_CHUNK,), jnp.int32),
            pltpu.VMEM((LANES,), jnp.int32),
            pltpu.VMEM_SHARED((NS, LO_BUCKETS), jnp.int32),
        ],
    )(flat2d, hvec)

    out = pl.pallas_call(
        _tc_mask_body,
        grid=(SC_GRID // TC_ROWS,),
        out_shape=jax.ShapeDtypeStruct((SC_GRID, SC_BLK), jnp.float32),
        in_specs=[
            pl.BlockSpec((NC, LO_BUCKETS // 128, 128), lambda i: (0, 0, 0)),
            pl.BlockSpec((8, 128), lambda i: (0, 0)),
            pl.BlockSpec((TC_ROWS, SC_BLK), lambda i: (i, 0)),
        ],
        out_specs=pl.BlockSpec((TC_ROWS, SC_BLK), lambda i: (i, 0)),
        scratch_shapes=[pltpu.SMEM((1,), jnp.int32)],
    )(hist_lo.reshape(NC, LO_BUCKETS // 128, 128), aux, flat2d)

    return out.reshape(B, L, D)


# speedup vs baseline: 27.5190x; 27.5190x over previous
import jax
import jax.numpy as jnp
from jax import lax
from jax.experimental import pallas as pl
from jax.experimental.pallas import tpu as pltpu
from jax.experimental.pallas import tpu_sc as plsc

TOPK = 131072
B, L, D = 512, 12, 4096
N = B * L * D
LANES = 16
NC, NS = 2, 16

NW = NC * NS

HI_BUCKETS = 32768
LO_BUCKETS = 65536

SC_BLK = 16384
SC_GRID = N // SC_BLK
TC_ROWS = 64


def _sc_mesh():
    return plsc.VectorSubcoreMesh(core_axis_name="c", subcore_axis_name="s")


_SC_PARAMS = pltpu.CompilerParams(needs_layout_passes=False)


def _zero_hist(hist_v, nwords):
    zeros16 = jnp.zeros((LANES,), jnp.int32)

    @pl.loop(0, nwords // LANES)
    def _(i):
        hist_v[pl.ds(i * LANES, LANES)] = zeros16


def _sc_hist_hi_body(flat_hbm, hist_hbm, hist_v):
    c = lax.axis_index("c")
    s = lax.axis_index("s")
    ones16 = jnp.ones((LANES,), jnp.int32)
    lane = lax.iota(jnp.int32, LANES)
    zoff = lane + HI_BUCKETS

    _zero_hist(hist_v, HI_BUCKETS + LANES)

    def body(in_vmem):
        @pl.loop(0, SC_BLK // LANES)
        def _(j):
            v = in_vmem[0, pl.ds(j * LANES, LANES)]
            bits = plsc.bitcast(v, jnp.int32)
            y = jnp.maximum(bits, 0)
            hi = lax.shift_right_logical(y, 16)
            hi2 = jnp.where(hi == 0, zoff, hi)
            plsc.addupdate_scatter(hist_v, [hi2], ones16)

    pltpu.emit_pipeline(
        body,
        grid=(SC_GRID,),
        in_specs=[pl.BlockSpec((1, SC_BLK), lambda i: (i, 0))],
        out_specs=[],
        core_axis_name=("c", "s"),
        dimension_semantics=(pltpu.PARALLEL,),
    )(flat_hbm)

    z = hist_v[pl.ds(HI_BUCKETS, LANES)]
    zsum = jnp.sum(z)
    zvec = jnp.where(lane == 0, zsum, 0)
    sl0 = pl.ds(0, LANES)
    hist_v[sl0] = hist_v[sl0] + zvec

    w = c * NS + s
    pltpu.sync_copy(hist_v.at[pl.ds(0, HI_BUCKETS)], hist_hbm.at[w])


def _sc_hist_lo_body(flat_hbm, hvec_hbm, hist_hbm, hist_v, hvec_v):
    c = lax.axis_index("c")
    s = lax.axis_index("s")
    ones16 = jnp.ones((LANES,), jnp.int32)

    _zero_hist(hist_v, LO_BUCKETS)
    pltpu.sync_copy(hvec_hbm, hvec_v)
    hstar = hvec_v[...]

    def body(in_vmem):
        @pl.loop(0, SC_BLK // LANES)
        def _(j):
            v = in_vmem[0, pl.ds(j * LANES, LANES)]
            bits = plsc.bitcast(v, jnp.int32)
            y = jnp.maximum(bits, 0)
            hi = lax.shift_right_logical(y, 16)
            lo = jnp.bitwise_and(y, 0xFFFF)
            mask = hi == hstar
            plsc.addupdate_scatter(hist_v, [lo], ones16, mask=mask)

    pltpu.emit_pipeline(
        body,
        grid=(SC_GRID,),
        in_specs=[pl.BlockSpec((1, SC_BLK), lambda i: (i, 0))],
        out_specs=[],
        core_axis_name=("c", "s"),
        dimension_semantics=(pltpu.PARALLEL,),
    )(flat_hbm)

    w = c * NS + s
    pltpu.sync_copy(hist_v, hist_hbm.at[w])


def _suffix_search(h, nbuckets, k, steps):
    rows = nbuckets // 128
    row = lax.broadcasted_iota(jnp.int32, (rows, 128), 0)
    col = lax.broadcasted_iota(jnp.int32, (rows, 128), 1)
    idx = row * 128 + col

    def cnt_ge(m):
        return jnp.sum(jnp.where(idx >= m, h, 0))

    def step(_, lohi):
        lo, hi = lohi
        mid = (lo + hi) // 2
        ok = cnt_ge(mid) >= k
        return jnp.where(ok, mid, lo), jnp.where(ok, hi, mid)

    lo, _ = lax.fori_loop(0, steps, step, (jnp.int32(0), jnp.int32(nbuckets)))
    above = jnp.sum(jnp.where(idx > lo, h, 0))
    return lo, above


def _tc_find_hi_body(hist_ref, aux_ref):
    h = jnp.sum(hist_ref[...], axis=0)
    hstar, above = _suffix_search(h, HI_BUCKETS, TOPK, 15)
    k2 = TOPK - above
    r = lax.broadcasted_iota(jnp.int32, (8, 128), 0)
    aux_ref[...] = jnp.where(r == 0, hstar, jnp.where(r == 1, k2, 0))


def _tc_mask_body(hist_ref, aux_ref, x_ref, o_ref, t_ref):
    @pl.when(pl.program_id(0) == 0)
    def _():
        hstar = aux_ref[0, 0]
        k2 = aux_ref[1, 0]
        h = jnp.sum(hist_ref[...], axis=0)
        lostar, _ = _suffix_search(h, LO_BUCKETS, k2, 16)
        t_ref[0] = jnp.bitwise_or(lax.shift_left(hstar, 16), lostar)

    t = t_ref[0]
    bits = lax.bitcast_convert_type(x_ref[...], jnp.int32)
    y = jnp.maximum(bits, 0)
    o_ref[...] = lax.bitcast_convert_type(jnp.where(y >= t, y, 0), jnp.float32)


@jax.jit
def kernel(features):
    flat2d = features.reshape(SC_GRID, SC_BLK)

    hist_hi = pl.kernel(
        _sc_hist_hi_body,
        out_type=jax.ShapeDtypeStruct((NW, HI_BUCKETS), jnp.int32),
        mesh=_sc_mesh(),
        compiler_params=_SC_PARAMS,
        scratch_types=[
            pltpu.VMEM((HI_BUCKETS + LANES,), jnp.int32),
        ],
    )(flat2d)

    aux = pl.pallas_call(
        _tc_find_hi_body,
        out_shape=jax.ShapeDtypeStruct((8, 128), jnp.int32),
        in_specs=[pl.BlockSpec((NW, HI_BUCKETS // 128, 128),
                               lambda: (0, 0, 0))],
        out_specs=pl.BlockSpec((8, 128), lambda: (0, 0)),
    )(hist_hi.reshape(NW, HI_BUCKETS // 128, 128))

    hvec = aux[0, :LANES]

    hist_lo = pl.kernel(
        _sc_hist_lo_body,
        out_type=jax.ShapeDtypeStruct((NW, LO_BUCKETS), jnp.int32),
        mesh=_sc_mesh(),
        compiler_params=_SC_PARAMS,
        scratch_types=[
            pltpu.VMEM((LO_BUCKETS,), jnp.int32),
            pltpu.VMEM((LANES,), jnp.int32),
        ],
    )(flat2d, hvec)

    out = pl.pallas_call(
        _tc_mask_body,
        grid=(SC_GRID // TC_ROWS,),
        out_shape=jax.ShapeDtypeStruct((SC_GRID, SC_BLK), jnp.float32),
        in_specs=[
            pl.BlockSpec((NW, LO_BUCKETS // 128, 128), lambda i: (0, 0, 0)),
            pl.BlockSpec((8, 128), lambda i: (0, 0)),
            pl.BlockSpec((TC_ROWS, SC_BLK), lambda i: (i, 0)),
        ],
        out_specs=pl.BlockSpec((TC_ROWS, SC_BLK), lambda i: (i, 0)),
        scratch_shapes=[pltpu.SMEM((1,), jnp.int32)],
    )(hist_lo.reshape(NW, LO_BUCKETS // 128, 128), aux, flat2d)

    return out.reshape(B, L, D)


# speedup vs baseline: 30.1990x; 1.0974x over previous
import jax
import jax.numpy as jnp
from jax import lax
from jax.experimental import pallas as pl
from jax.experimental.pallas import tpu as pltpu
from jax.experimental.pallas import tpu_sc as plsc

TOPK = 131072
B, L, D = 512, 12, 4096
N = B * L * D
LANES = 16
NC, NS = 2, 16

NW = NC * NS

HI_BUCKETS = 32768
LO_BUCKETS = 65536

SC_BLK = 16384
SC_GRID = N // SC_BLK
TC_ROWS = 64


def _sc_mesh():
    return plsc.VectorSubcoreMesh(core_axis_name="c", subcore_axis_name="s")


_SC_PARAMS = pltpu.CompilerParams(needs_layout_passes=False)


def _zero_hist(hist_v, nwords):
    zeros16 = jnp.zeros((LANES,), jnp.int32)

    @pl.loop(0, nwords // LANES)
    def _(i):
        hist_v[pl.ds(i * LANES, LANES)] = zeros16


def _sc_hist_hi_body(flat_hbm, hist_hbm, hist_v):
    c = lax.axis_index("c")
    s = lax.axis_index("s")
    ones16 = jnp.ones((LANES,), jnp.int32)
    lane = lax.iota(jnp.int32, LANES)
    zoff = lane + HI_BUCKETS

    _zero_hist(hist_v, HI_BUCKETS + LANES)

    def body(in_vmem):
        @pl.loop(0, SC_BLK // LANES, unroll=8)
        def _(j):
            v = in_vmem[0, pl.ds(j * LANES, LANES)]
            bits = plsc.bitcast(v, jnp.int32)
            y = jnp.maximum(bits, 0)
            hi = lax.shift_right_logical(y, 16)
            hi2 = jnp.where(hi == 0, zoff, hi)
            plsc.addupdate_scatter(hist_v, [hi2], ones16)

    pltpu.emit_pipeline(
        body,
        grid=(SC_GRID,),
        in_specs=[pl.BlockSpec((1, SC_BLK), lambda i: (i, 0))],
        out_specs=[],
        core_axis_name=("c", "s"),
        dimension_semantics=(pltpu.PARALLEL,),
    )(flat_hbm)

    z = hist_v[pl.ds(HI_BUCKETS, LANES)]
    zsum = jnp.sum(z)
    zvec = jnp.where(lane == 0, zsum, 0)
    sl0 = pl.ds(0, LANES)
    hist_v[sl0] = hist_v[sl0] + zvec

    w = c * NS + s
    pltpu.sync_copy(hist_v.at[pl.ds(0, HI_BUCKETS)], hist_hbm.at[w])


def _sc_hist_lo_body(flat_hbm, hvec_hbm, hist_hbm, hist_v, hvec_v):
    c = lax.axis_index("c")
    s = lax.axis_index("s")
    ones16 = jnp.ones((LANES,), jnp.int32)

    _zero_hist(hist_v, LO_BUCKETS)
    pltpu.sync_copy(hvec_hbm, hvec_v)
    hstar = hvec_v[...]

    def body(in_vmem):
        @pl.loop(0, SC_BLK // LANES, unroll=8)
        def _(j):
            v = in_vmem[0, pl.ds(j * LANES, LANES)]
            bits = plsc.bitcast(v, jnp.int32)
            y = jnp.maximum(bits, 0)
            hi = lax.shift_right_logical(y, 16)
            lo = jnp.bitwise_and(y, 0xFFFF)
            mask = hi == hstar
            plsc.addupdate_scatter(hist_v, [lo], ones16, mask=mask)

    pltpu.emit_pipeline(
        body,
        grid=(SC_GRID,),
        in_specs=[pl.BlockSpec((1, SC_BLK), lambda i: (i, 0))],
        out_specs=[],
        core_axis_name=("c", "s"),
        dimension_semantics=(pltpu.PARALLEL,),
    )(flat_hbm)

    w = c * NS + s
    pltpu.sync_copy(hist_v, hist_hbm.at[w])


def _suffix_search(h, nbuckets, k, steps):
    rows = nbuckets // 128
    row = lax.broadcasted_iota(jnp.int32, (rows, 128), 0)
    col = lax.broadcasted_iota(jnp.int32, (rows, 128), 1)
    idx = row * 128 + col

    def cnt_ge(m):
        return jnp.sum(jnp.where(idx >= m, h, 0))

    def step(_, lohi):
        lo, hi = lohi
        mid = (lo + hi) // 2
        ok = cnt_ge(mid) >= k
        return jnp.where(ok, mid, lo), jnp.where(ok, hi, mid)

    lo, _ = lax.fori_loop(0, steps, step, (jnp.int32(0), jnp.int32(nbuckets)))
    above = jnp.sum(jnp.where(idx > lo, h, 0))
    return lo, above


def _tc_find_hi_body(hist_ref, aux_ref):
    h = jnp.sum(hist_ref[...], axis=0)
    hstar, above = _suffix_search(h, HI_BUCKETS, TOPK, 15)
    k2 = TOPK - above
    r = lax.broadcasted_iota(jnp.int32, (8, 128), 0)
    aux_ref[...] = jnp.where(r == 0, hstar, jnp.where(r == 1, k2, 0))


def _tc_mask_body(hist_ref, aux_ref, x_ref, o_ref, t_ref):
    @pl.when(pl.program_id(0) == 0)
    def _():
        hstar = aux_ref[0, 0]
        k2 = aux_ref[1, 0]
        h = jnp.sum(hist_ref[...], axis=0)
        lostar, _ = _suffix_search(h, LO_BUCKETS, k2, 16)
        t_ref[0] = jnp.bitwise_or(lax.shift_left(hstar, 16), lostar)

    t = t_ref[0]
    bits = lax.bitcast_convert_type(x_ref[...], jnp.int32)
    y = jnp.maximum(bits, 0)
    o_ref[...] = lax.bitcast_convert_type(jnp.where(y >= t, y, 0), jnp.float32)


@jax.jit
def kernel(features):
    flat2d = features.reshape(SC_GRID, SC_BLK)

    hist_hi = pl.kernel(
        _sc_hist_hi_body,
        out_type=jax.ShapeDtypeStruct((NW, HI_BUCKETS), jnp.int32),
        mesh=_sc_mesh(),
        compiler_params=_SC_PARAMS,
        scratch_types=[
            pltpu.VMEM((HI_BUCKETS + LANES,), jnp.int32),
        ],
    )(flat2d)

    aux = pl.pallas_call(
        _tc_find_hi_body,
        out_shape=jax.ShapeDtypeStruct((8, 128), jnp.int32),
        in_specs=[pl.BlockSpec((NW, HI_BUCKETS // 128, 128),
                               lambda: (0, 0, 0))],
        out_specs=pl.BlockSpec((8, 128), lambda: (0, 0)),
    )(hist_hi.reshape(NW, HI_BUCKETS // 128, 128))

    hvec = aux[0, :LANES]

    hist_lo = pl.kernel(
        _sc_hist_lo_body,
        out_type=jax.ShapeDtypeStruct((NW, LO_BUCKETS), jnp.int32),
        mesh=_sc_mesh(),
        compiler_params=_SC_PARAMS,
        scratch_types=[
            pltpu.VMEM((LO_BUCKETS,), jnp.int32),
            pltpu.VMEM((LANES,), jnp.int32),
        ],
    )(flat2d, hvec)

    out = pl.pallas_call(
        _tc_mask_body,
        grid=(SC_GRID // TC_ROWS,),
        out_shape=jax.ShapeDtypeStruct((SC_GRID, SC_BLK), jnp.float32),
        in_specs=[
            pl.BlockSpec((NW, LO_BUCKETS // 128, 128), lambda i: (0, 0, 0)),
            pl.BlockSpec((8, 128), lambda i: (0, 0)),
            pl.BlockSpec((TC_ROWS, SC_BLK), lambda i: (i, 0)),
        ],
        out_specs=pl.BlockSpec((TC_ROWS, SC_BLK), lambda i: (i, 0)),
        scratch_shapes=[pltpu.SMEM((1,), jnp.int32)],
    )(hist_lo.reshape(NW, LO_BUCKETS // 128, 128), aux, flat2d)

    return out.reshape(B, L, D)


# speedup vs baseline: 66.9886x; 2.2182x over previous
import jax
import jax.numpy as jnp
from jax import lax
from jax.experimental import pallas as pl
from jax.experimental.pallas import tpu as pltpu
from jax.experimental.pallas import tpu_sc as plsc

TOPK = 131072
B, L, D = 512, 12, 4096
N = B * L * D
LANES = 16
NC, NS = 2, 16

NW = NC * NS

HI_BUCKETS = 32768
LO_BUCKETS = 65536

SC_BLK = 16384
SC_GRID = N // SC_BLK
TC_ROWS = 64


def _sc_mesh():
    return plsc.VectorSubcoreMesh(core_axis_name="c", subcore_axis_name="s")


_SC_PARAMS = pltpu.CompilerParams(needs_layout_passes=False)


def _zero_hist(hist_v, nwords):
    zeros16 = jnp.zeros((LANES,), jnp.int32)

    @pl.loop(0, nwords // LANES)
    def _(i):
        hist_v[pl.ds(i * LANES, LANES)] = zeros16


def _sc_hist_hi_body(flat_hbm, hist_hbm, hist_v):
    c = lax.axis_index("c")
    s = lax.axis_index("s")
    ones16 = jnp.ones((LANES,), jnp.int32)
    lane = lax.iota(jnp.int32, LANES)
    zoff = lane + HI_BUCKETS

    _zero_hist(hist_v, HI_BUCKETS + LANES)

    def body(in_vmem):
        @plsc.parallel_loop(0, SC_BLK // LANES, unroll=8)
        def _(j):
            v = in_vmem[0, pl.ds(j * LANES, LANES)]
            bits = plsc.bitcast(v, jnp.int32)
            y = jnp.maximum(bits, 0)
            hi = lax.shift_right_logical(y, 16)
            hi2 = jnp.where(hi == 0, zoff, hi)
            plsc.addupdate_scatter(hist_v, [hi2], ones16)

    pltpu.emit_pipeline(
        body,
        grid=(SC_GRID,),
        in_specs=[pl.BlockSpec((1, SC_BLK), lambda i: (i, 0))],
        out_specs=[],
        core_axis_name=("c", "s"),
        dimension_semantics=(pltpu.PARALLEL,),
    )(flat_hbm)

    z = hist_v[pl.ds(HI_BUCKETS, LANES)]
    zsum = jnp.sum(z)
    zvec = jnp.where(lane == 0, zsum, 0)
    sl0 = pl.ds(0, LANES)
    hist_v[sl0] = hist_v[sl0] + zvec

    w = c * NS + s
    pltpu.sync_copy(hist_v.at[pl.ds(0, HI_BUCKETS)], hist_hbm.at[w])


def _sc_hist_lo_body(flat_hbm, hvec_hbm, hist_hbm, hist_v, hvec_v):
    c = lax.axis_index("c")
    s = lax.axis_index("s")
    ones16 = jnp.ones((LANES,), jnp.int32)

    _zero_hist(hist_v, LO_BUCKETS)
    pltpu.sync_copy(hvec_hbm, hvec_v)
    hstar = hvec_v[...]

    def body(in_vmem):
        @plsc.parallel_loop(0, SC_BLK // LANES, unroll=8)
        def _(j):
            v = in_vmem[0, pl.ds(j * LANES, LANES)]
            bits = plsc.bitcast(v, jnp.int32)
            y = jnp.maximum(bits, 0)
            hi = lax.shift_right_logical(y, 16)
            lo = jnp.bitwise_and(y, 0xFFFF)
            mask = hi == hstar
            plsc.addupdate_scatter(hist_v, [lo], ones16, mask=mask)

    pltpu.emit_pipeline(
        body,
        grid=(SC_GRID,),
        in_specs=[pl.BlockSpec((1, SC_BLK), lambda i: (i, 0))],
        out_specs=[],
        core_axis_name=("c", "s"),
        dimension_semantics=(pltpu.PARALLEL,),
    )(flat_hbm)

    w = c * NS + s
    pltpu.sync_copy(hist_v, hist_hbm.at[w])


def _suffix_search(h, nbuckets, k, steps):
    rows = nbuckets // 128
    row = lax.broadcasted_iota(jnp.int32, (rows, 128), 0)
    col = lax.broadcasted_iota(jnp.int32, (rows, 128), 1)
    idx = row * 128 + col

    def cnt_ge(m):
        return jnp.sum(jnp.where(idx >= m, h, 0))

    def step(_, lohi):
        lo, hi = lohi
        mid = (lo + hi) // 2
        ok = cnt_ge(mid) >= k
        return jnp.where(ok, mid, lo), jnp.where(ok, hi, mid)

    lo, _ = lax.fori_loop(0, steps, step, (jnp.int32(0), jnp.int32(nbuckets)))
    above = jnp.sum(jnp.where(idx > lo, h, 0))
    return lo, above


def _tc_find_hi_body(hist_ref, aux_ref):
    h = jnp.sum(hist_ref[...], axis=0)
    hstar, above = _suffix_search(h, HI_BUCKETS, TOPK, 15)
    k2 = TOPK - above
    r = lax.broadcasted_iota(jnp.int32, (8, 128), 0)
    aux_ref[...] = jnp.where(r == 0, hstar, jnp.where(r == 1, k2, 0))


def _tc_mask_body(hist_ref, aux_ref, x_ref, o_ref, t_ref):
    @pl.when(pl.program_id(0) == 0)
    def _():
        hstar = aux_ref[0, 0]
        k2 = aux_ref[1, 0]
        h = jnp.sum(hist_ref[...], axis=0)
        lostar, _ = _suffix_search(h, LO_BUCKETS, k2, 16)
        t_ref[0] = jnp.bitwise_or(lax.shift_left(hstar, 16), lostar)

    t = t_ref[0]
    bits = lax.bitcast_convert_type(x_ref[...], jnp.int32)
    y = jnp.maximum(bits, 0)
    o_ref[...] = lax.bitcast_convert_type(jnp.where(y >= t, y, 0), jnp.float32)


@jax.jit
def kernel(features):
    flat2d = features.reshape(SC_GRID, SC_BLK)

    hist_hi = pl.kernel(
        _sc_hist_hi_body,
        out_type=jax.ShapeDtypeStruct((NW, HI_BUCKETS), jnp.int32),
        mesh=_sc_mesh(),
        compiler_params=_SC_PARAMS,
        scratch_types=[
            pltpu.VMEM((HI_BUCKETS + LANES,), jnp.int32),
        ],
    )(flat2d)

    aux = pl.pallas_call(
        _tc_find_hi_body,
        out_shape=jax.ShapeDtypeStruct((8, 128), jnp.int32),
        in_specs=[pl.BlockSpec((NW, HI_BUCKETS // 128, 128),
                               lambda: (0, 0, 0))],
        out_specs=pl.BlockSpec((8, 128), lambda: (0, 0)),
    )(hist_hi.reshape(NW, HI_BUCKETS // 128, 128))

    hvec = aux[0, :LANES]

    hist_lo = pl.kernel(
        _sc_hist_lo_body,
        out_type=jax.ShapeDtypeStruct((NW, LO_BUCKETS), jnp.int32),
        mesh=_sc_mesh(),
        compiler_params=_SC_PARAMS,
        scratch_types=[
            pltpu.VMEM((LO_BUCKETS,), jnp.int32),
            pltpu.VMEM((LANES,), jnp.int32),
        ],
    )(flat2d, hvec)

    out = pl.pallas_call(
        _tc_mask_body,
        grid=(SC_GRID // TC_ROWS,),
        out_shape=jax.ShapeDtypeStruct((SC_GRID, SC_BLK), jnp.float32),
        in_specs=[
            pl.BlockSpec((NW, LO_BUCKETS // 128, 128), lambda i: (0, 0, 0)),
            pl.BlockSpec((8, 128), lambda i: (0, 0)),
            pl.BlockSpec((TC_ROWS, SC_BLK), lambda i: (i, 0)),
        ],
        out_specs=pl.BlockSpec((TC_ROWS, SC_BLK), lambda i: (i, 0)),
        scratch_shapes=[pltpu.SMEM((1,), jnp.int32)],
    )(hist_lo.reshape(NW, LO_BUCKETS // 128, 128), aux, flat2d)

    return out.reshape(B, L, D)


# speedup vs baseline: 72.9381x; 1.0888x over previous
import jax
import jax.numpy as jnp
from jax import lax
from jax.experimental import pallas as pl
from jax.experimental.pallas import tpu as pltpu
from jax.experimental.pallas import tpu_sc as plsc

TOPK = 131072
B, L, D = 512, 12, 4096
N = B * L * D
LANES = 16
NC, NS = 2, 16

NW = NC * NS

HI_BUCKETS = 32768
LO_BUCKETS = 65536

SC_BLK = 24576
SC_GRID = N // SC_BLK
TC_ROWS = 32


def _sc_mesh():
    return plsc.VectorSubcoreMesh(core_axis_name="c", subcore_axis_name="s")


_SC_PARAMS = pltpu.CompilerParams(needs_layout_passes=False)


def _zero_hist(hist_v, nwords):
    zeros16 = jnp.zeros((LANES,), jnp.int32)

    @pl.loop(0, nwords // LANES)
    def _(i):
        hist_v[pl.ds(i * LANES, LANES)] = zeros16


def _sc_hist_hi_body(flat_hbm, hist_hbm, hist_v):
    c = lax.axis_index("c")
    s = lax.axis_index("s")
    ones16 = jnp.ones((LANES,), jnp.int32)
    lane = lax.iota(jnp.int32, LANES)
    zoff = lane + HI_BUCKETS

    _zero_hist(hist_v, HI_BUCKETS + LANES)

    def body(in_vmem):
        @plsc.parallel_loop(0, SC_BLK // LANES, unroll=8)
        def _(j):
            v = in_vmem[0, pl.ds(j * LANES, LANES)]
            bits = plsc.bitcast(v, jnp.int32)
            y = jnp.maximum(bits, 0)
            hi = lax.shift_right_logical(y, 16)
            hi2 = jnp.where(hi == 0, zoff, hi)
            plsc.addupdate_scatter(hist_v, [hi2], ones16)

    pltpu.emit_pipeline(
        body,
        grid=(SC_GRID,),
        in_specs=[pl.BlockSpec((1, SC_BLK), lambda i: (i, 0))],
        out_specs=[],
        core_axis_name=("c", "s"),
        dimension_semantics=(pltpu.PARALLEL,),
    )(flat_hbm)

    z = hist_v[pl.ds(HI_BUCKETS, LANES)]
    zsum = jnp.sum(z)
    zvec = jnp.where(lane == 0, zsum, 0)
    sl0 = pl.ds(0, LANES)
    hist_v[sl0] = hist_v[sl0] + zvec

    w = c * NS + s
    pltpu.sync_copy(hist_v.at[pl.ds(0, HI_BUCKETS)], hist_hbm.at[w])


def _sc_hist_lo_body(flat_hbm, hvec_hbm, hist_hbm, hist_v, hvec_v):
    c = lax.axis_index("c")
    s = lax.axis_index("s")
    ones16 = jnp.ones((LANES,), jnp.int32)

    _zero_hist(hist_v, LO_BUCKETS)
    pltpu.sync_copy(hvec_hbm, hvec_v)
    hstar = hvec_v[...]

    def body(in_vmem):
        @plsc.parallel_loop(0, SC_BLK // LANES, unroll=8)
        def _(j):
            v = in_vmem[0, pl.ds(j * LANES, LANES)]
            bits = plsc.bitcast(v, jnp.int32)
            y = jnp.maximum(bits, 0)
            hi = lax.shift_right_logical(y, 16)
            lo = jnp.bitwise_and(y, 0xFFFF)
            mask = hi == hstar
            plsc.addupdate_scatter(hist_v, [lo], ones16, mask=mask)

    pltpu.emit_pipeline(
        body,
        grid=(SC_GRID,),
        in_specs=[pl.BlockSpec((1, SC_BLK), lambda i: (i, 0))],
        out_specs=[],
        core_axis_name=("c", "s"),
        dimension_semantics=(pltpu.PARALLEL,),
    )(flat_hbm)

    w = c * NS + s
    pltpu.sync_copy(hist_v, hist_hbm.at[w])


def _suffix_search(h, nbuckets, k, steps):
    rows = nbuckets // 128
    row = lax.broadcasted_iota(jnp.int32, (rows, 128), 0)
    col = lax.broadcasted_iota(jnp.int32, (rows, 128), 1)
    idx = row * 128 + col

    def cnt_ge(m):
        return jnp.sum(jnp.where(idx >= m, h, 0))

    def step(_, lohi):
        lo, hi = lohi
        mid = (lo + hi) // 2
        ok = cnt_ge(mid) >= k
        return jnp.where(ok, mid, lo), jnp.where(ok, hi, mid)

    lo, _ = lax.fori_loop(0, steps, step, (jnp.int32(0), jnp.int32(nbuckets)))
    above = jnp.sum(jnp.where(idx > lo, h, 0))
    return lo, above


def _tc_find_hi_body(hist_ref, aux_ref):
    h = jnp.sum(hist_ref[...], axis=0)
    hstar, above = _suffix_search(h, HI_BUCKETS, TOPK, 15)
    k2 = TOPK - above
    r = lax.broadcasted_iota(jnp.int32, (8, 128), 0)
    aux_ref[...] = jnp.where(r == 0, hstar, jnp.where(r == 1, k2, 0))


def _tc_mask_body(hist_ref, aux_ref, x_ref, o_ref, t_ref):
    @pl.when(pl.program_id(0) == 0)
    def _():
        hstar = aux_ref[0, 0]
        k2 = aux_ref[1, 0]
        h = jnp.sum(hist_ref[...], axis=0)
        lostar, _ = _suffix_search(h, LO_BUCKETS, k2, 16)
        t_ref[0] = jnp.bitwise_or(lax.shift_left(hstar, 16), lostar)

    t = t_ref[0]
    bits = lax.bitcast_convert_type(x_ref[...], jnp.int32)
    y = jnp.maximum(bits, 0)
    o_ref[...] = lax.bitcast_convert_type(jnp.where(y >= t, y, 0), jnp.float32)


@jax.jit
def kernel(features):
    flat2d = features.reshape(SC_GRID, SC_BLK)

    hist_hi = pl.kernel(
        _sc_hist_hi_body,
        out_type=jax.ShapeDtypeStruct((NW, HI_BUCKETS), jnp.int32),
        mesh=_sc_mesh(),
        compiler_params=_SC_PARAMS,
        scratch_types=[
            pltpu.VMEM((HI_BUCKETS + LANES,), jnp.int32),
        ],
    )(flat2d)

    aux = pl.pallas_call(
        _tc_find_hi_body,
        out_shape=jax.ShapeDtypeStruct((8, 128), jnp.int32),
        in_specs=[pl.BlockSpec((NW, HI_BUCKETS // 128, 128),
                               lambda: (0, 0, 0))],
        out_specs=pl.BlockSpec((8, 128), lambda: (0, 0)),
    )(hist_hi.reshape(NW, HI_BUCKETS // 128, 128))

    hvec = aux[0, :LANES]

    hist_lo = pl.kernel(
        _sc_hist_lo_body,
        out_type=jax.ShapeDtypeStruct((NW, LO_BUCKETS), jnp.int32),
        mesh=_sc_mesh(),
        compiler_params=_SC_PARAMS,
        scratch_types=[
            pltpu.VMEM((LO_BUCKETS,), jnp.int32),
            pltpu.VMEM((LANES,), jnp.int32),
        ],
    )(flat2d, hvec)

    out = pl.pallas_call(
        _tc_mask_body,
        grid=(B // TC_ROWS,),
        out_shape=jax.ShapeDtypeStruct((B, L, D), jnp.float32),
        in_specs=[
            pl.BlockSpec((NW, LO_BUCKETS // 128, 128), lambda i: (0, 0, 0)),
            pl.BlockSpec((8, 128), lambda i: (0, 0)),
            pl.BlockSpec((TC_ROWS, L, D), lambda i: (i, 0, 0)),
        ],
        out_specs=pl.BlockSpec((TC_ROWS, L, D), lambda i: (i, 0, 0)),
        scratch_shapes=[pltpu.SMEM((1,), jnp.int32)],
    )(hist_lo.reshape(NW, LO_BUCKETS // 128, 128), aux, features)

    return out


# speedup vs baseline: 149.9741x; 2.0562x over previous
import jax
import jax.numpy as jnp
from jax import lax
from jax.experimental import pallas as pl
from jax.experimental.pallas import tpu as pltpu
from jax.experimental.pallas import tpu_sc as plsc

TOPK = 131072
B, L, D = 512, 12, 4096
N = B * L * D
LANES = 16
NC, NS = 2, 16

NW = NC * NS

HI_BUCKETS = 32768
LO_BUCKETS = 65536

ROWS2D, COLS2D = 6144, 4096
SC_R_HI = 8
LO_COLS = 2048
TC_ROWS = 32


def _sc_mesh():
    return plsc.VectorSubcoreMesh(core_axis_name="c", subcore_axis_name="s")


_SC_PARAMS = pltpu.CompilerParams(needs_layout_passes=False)


def _zero_hist(hist_v, nwords):
    zeros16 = jnp.zeros((LANES,), jnp.int32)

    @pl.loop(0, nwords // LANES)
    def _(i):
        hist_v[pl.ds(i * LANES, LANES)] = zeros16


def _sc_hist_hi_body(flat_hbm, hist_hbm, hist_v):
    c = lax.axis_index("c")
    s = lax.axis_index("s")
    ones16 = jnp.ones((LANES,), jnp.int32)
    lane = lax.iota(jnp.int32, LANES)
    zoff = lane + HI_BUCKETS

    _zero_hist(hist_v, HI_BUCKETS + LANES)

    def body(in_vmem):
        @pl.loop(0, SC_R_HI)
        def _(r):
            @plsc.parallel_loop(0, COLS2D // LANES, unroll=8)
            def _(j):
                v = in_vmem[r, pl.ds(j * LANES, LANES)]
                bits = plsc.bitcast(v, jnp.int32)
                y = jnp.maximum(bits, 0)
                hi = lax.shift_right_logical(y, 16)
                hi2 = jnp.where(hi == 0, zoff, hi)
                plsc.addupdate_scatter(hist_v, [hi2], ones16)

    pltpu.emit_pipeline(
        body,
        grid=(ROWS2D // SC_R_HI,),
        in_specs=[pl.BlockSpec((SC_R_HI, COLS2D), lambda i: (i, 0))],
        out_specs=[],
        core_axis_name=("c", "s"),
        dimension_semantics=(pltpu.PARALLEL,),
    )(flat_hbm)

    z = hist_v[pl.ds(HI_BUCKETS, LANES)]
    zsum = jnp.sum(z)
    zvec = jnp.where(lane == 0, zsum, 0)
    sl0 = pl.ds(0, LANES)
    hist_v[sl0] = hist_v[sl0] + zvec

    w = c * NS + s
    pltpu.sync_copy(hist_v.at[pl.ds(0, HI_BUCKETS)], hist_hbm.at[w])


def _sc_hist_lo_body(flat_hbm, hvec_hbm, hist_hbm, hist_v, hvec_v):
    c = lax.axis_index("c")
    s = lax.axis_index("s")
    ones16 = jnp.ones((LANES,), jnp.int32)

    _zero_hist(hist_v, LO_BUCKETS)
    pltpu.sync_copy(hvec_hbm, hvec_v)
    hstar = hvec_v[...]

    def body(in_vmem):
        @pl.loop(0, 8)
        def _(r):
            @plsc.parallel_loop(0, LO_COLS // LANES, unroll=8)
            def _(j):
                v = in_vmem[r, pl.ds(j * LANES, LANES)]
                bits = plsc.bitcast(v, jnp.int32)
                y = jnp.maximum(bits, 0)
                hi = lax.shift_right_logical(y, 16)
                lo = jnp.bitwise_and(y, 0xFFFF)
                mask = hi == hstar
                plsc.addupdate_scatter(hist_v, [lo], ones16, mask=mask)

    pltpu.emit_pipeline(
        body,
        grid=(ROWS2D // 8, COLS2D // LO_COLS),
        in_specs=[pl.BlockSpec((8, LO_COLS), lambda i, j: (i, j))],
        out_specs=[],
        core_axis_name=("c", "s"),
        dimension_semantics=(pltpu.PARALLEL, pltpu.PARALLEL),
    )(flat_hbm)

    w = c * NS + s
    pltpu.sync_copy(hist_v, hist_hbm.at[w])


def _suffix_search(h, nbuckets, k, steps):
    rows = nbuckets // 128
    row = lax.broadcasted_iota(jnp.int32, (rows, 128), 0)
    col = lax.broadcasted_iota(jnp.int32, (rows, 128), 1)
    idx = row * 128 + col

    def cnt_ge(m):
        return jnp.sum(jnp.where(idx >= m, h, 0))

    def step(_, lohi):
        lo, hi = lohi
        mid = (lo + hi) // 2
        ok = cnt_ge(mid) >= k
        return jnp.where(ok, mid, lo), jnp.where(ok, hi, mid)

    lo, _ = lax.fori_loop(0, steps, step, (jnp.int32(0), jnp.int32(nbuckets)))
    above = jnp.sum(jnp.where(idx > lo, h, 0))
    return lo, above


def _tc_find_hi_body(hist_ref, aux_ref):
    h = jnp.sum(hist_ref[...], axis=0)
    hstar, above = _suffix_search(h, HI_BUCKETS, TOPK, 15)
    k2 = TOPK - above
    r = lax.broadcasted_iota(jnp.int32, (8, 128), 0)
    aux_ref[...] = jnp.where(r == 0, hstar, jnp.where(r == 1, k2, 0))


def _tc_mask_body(hist_ref, aux_ref, x_ref, o_ref, t_ref):
    @pl.when(pl.program_id(0) == 0)
    def _():
        hstar = aux_ref[0, 0]
        k2 = aux_ref[1, 0]
        h = jnp.sum(hist_ref[...], axis=0)
        lostar, _ = _suffix_search(h, LO_BUCKETS, k2, 16)
        t_ref[0] = jnp.bitwise_or(lax.shift_left(hstar, 16), lostar)

    t = t_ref[0]
    bits = lax.bitcast_convert_type(x_ref[...], jnp.int32)
    y = jnp.maximum(bits, 0)
    o_ref[...] = lax.bitcast_convert_type(jnp.where(y >= t, y, 0), jnp.float32)


@jax.jit
def kernel(features):
    xt = jnp.transpose(features, (1, 0, 2))
    flat2d = xt.reshape(ROWS2D, COLS2D)

    hist_hi = pl.kernel(
        _sc_hist_hi_body,
        out_type=jax.ShapeDtypeStruct((NW, HI_BUCKETS), jnp.int32),
        mesh=_sc_mesh(),
        compiler_params=_SC_PARAMS,
        scratch_types=[
            pltpu.VMEM((HI_BUCKETS + LANES,), jnp.int32),
        ],
    )(flat2d)

    aux = pl.pallas_call(
        _tc_find_hi_body,
        out_shape=jax.ShapeDtypeStruct((8, 128), jnp.int32),
        in_specs=[pl.BlockSpec((NW, HI_BUCKETS // 128, 128),
                               lambda: (0, 0, 0))],
        out_specs=pl.BlockSpec((8, 128), lambda: (0, 0)),
    )(hist_hi.reshape(NW, HI_BUCKETS // 128, 128))

    hvec = aux[0, :LANES]

    hist_lo = pl.kernel(
        _sc_hist_lo_body,
        out_type=jax.ShapeDtypeStruct((NW, LO_BUCKETS), jnp.int32),
        mesh=_sc_mesh(),
        compiler_params=_SC_PARAMS,
        scratch_types=[
            pltpu.VMEM((LO_BUCKETS,), jnp.int32),
            pltpu.VMEM((LANES,), jnp.int32),
        ],
    )(flat2d, hvec)

    out_t = pl.pallas_call(
        _tc_mask_body,
        grid=(B // TC_ROWS,),
        out_shape=jax.ShapeDtypeStruct((L, B, D), jnp.float32),
        in_specs=[
            pl.BlockSpec((NW, LO_BUCKETS // 128, 128), lambda i: (0, 0, 0)),
            pl.BlockSpec((8, 128), lambda i: (0, 0)),
            pl.BlockSpec((L, TC_ROWS, D), lambda i: (0, i, 0)),
        ],
        out_specs=pl.BlockSpec((L, TC_ROWS, D), lambda i: (0, i, 0)),
        scratch_shapes=[pltpu.SMEM((1,), jnp.int32)],
    )(hist_lo.reshape(NW, LO_BUCKETS // 128, 128), aux, xt)

    return jnp.transpose(out_t, (1, 0, 2))
